# baseline (speedup 1.0000x reference)
import functools

import jax
import jax.numpy as jnp
from jax import lax
from jax.experimental import pallas as pl
from jax.experimental.pallas import tpu as pltpu
from jax.experimental.pallas import tpu_sc as plsc

_VOCAB = 100000
_D = 128
_B = 4096
_L = 200
_N = _B * _L


_ROWS_PER_BLOCK = 10000
_N_BLOCKS = _VOCAB // _ROWS_PER_BLOCK


def _proj_body(t_ref, w_ref, o_ref):
    o_ref[...] = lax.dot_general(
        t_ref[...], w_ref[...],
        (((1,), (1,)), ((), ())),
        preferred_element_type=jnp.float32,
    )


def _project(table, W):
    return pl.pallas_call(
        _proj_body,
        grid=(_N_BLOCKS,),
        in_specs=[
            pl.BlockSpec((_ROWS_PER_BLOCK, _D), lambda i: (i, 0)),
            pl.BlockSpec((_D, _D), lambda i: (0, 0)),
        ],
        out_specs=pl.BlockSpec((_ROWS_PER_BLOCK, _D), lambda i: (i, 0)),
        out_shape=jax.ShapeDtypeStruct((_VOCAB, _D), jnp.float32),
    )(table, W)


_NC = 2
_NS = 16
_NW = _NC * _NS
_PER_W = _N // _NW
_C = 200
_ITERS = _PER_W // _C
_NBUF = 4
_LEAD = 1

_mesh = plsc.VectorSubcoreMesh(
    core_axis_name="c", subcore_axis_name="s",
    num_cores=_NC, num_subcores=_NS,
)


@functools.partial(
    pl.kernel,
    out_type=jax.ShapeDtypeStruct((_N, _D), jnp.float32),
    mesh=_mesh,
    scratch_types=[
        pltpu.VMEM((_PER_W,), jnp.int32),
        [pltpu.VMEM((_C, _D), jnp.float32) for _ in range(_NBUF)],
        [pltpu.SemaphoreType.DMA for _ in range(_NBUF)],
        [pltpu.SemaphoreType.DMA for _ in range(_NBUF)],
    ],
)
def _gather(tbl_hbm, idx_hbm, out_hbm, idx_all, rows, gsem, wsem):
    wid = lax.axis_index("s") * _NC + lax.axis_index("c")
    base = wid * _PER_W

    pltpu.sync_copy(idx_hbm.at[pl.ds(base, _PER_W)], idx_all)

    def g_copy(s, b):
        return pltpu.make_async_copy(
            tbl_hbm.at[idx_all.at[pl.ds(s * _C, _C)]], rows[b], gsem[b])

    def w_copy(s, b):
        off = pl.multiple_of(base + s * _C, _C)
        return pltpu.make_async_copy(rows[b], out_hbm.at[pl.ds(off, _C)],
                                     wsem[b])

    for b in range(_NBUF):
        g_copy(b, b).start()
        if b >= _LEAD:
            g_copy(b - _LEAD, b - _LEAD).wait()
            w_copy(b - _LEAD, b - _LEAD).start()

    def round_body(j, carry):
        for b in range(_NBUF):
            s = j * _NBUF + b
            w_copy(s - _NBUF, b).wait()
            g_copy(s, b).start()
            b2 = (b + _NBUF - _LEAD) % _NBUF
            g_copy(s - _LEAD, b2).wait()
            w_copy(s - _LEAD, b2).start()
        return carry

    lax.fori_loop(1, _ITERS // _NBUF, round_body, 0)

    for s in range(_ITERS - _LEAD, _ITERS):
        b = s % _NBUF
        g_copy(s, b).wait()
        w_copy(s, b).start()
    for s in range(_ITERS - _NBUF, _ITERS):
        w_copy(s, s % _NBUF).wait()


def kernel(x, table, W):
    P = _project(table, W)
    idx = x.reshape(_N).astype(jnp.int32)
    out = _gather(P, idx)
    return out.reshape(_B, _L, _D)


# speedup vs baseline: 1.0022x; 1.0022x over previous
import functools

import jax
import jax.numpy as jnp
from jax import lax
from jax.experimental import pallas as pl
from jax.experimental.pallas import tpu as pltpu
from jax.experimental.pallas import tpu_sc as plsc

_VOCAB = 100000
_D = 128
_B = 4096
_L = 200
_N = _B * _L


_ROWS_PER_BLOCK = 25000
_N_BLOCKS = _VOCAB // _ROWS_PER_BLOCK


def _proj_body(t_ref, w_ref, o_ref):
    o_ref[...] = lax.dot_general(
        t_ref[...], w_ref[...],
        (((1,), (1,)), ((), ())),
        preferred_element_type=jnp.float32,
    )


def _project(table, W):
    return pl.pallas_call(
        _proj_body,
        grid=(_N_BLOCKS,),
        in_specs=[
            pl.BlockSpec((_ROWS_PER_BLOCK, _D), lambda i: (i, 0)),
            pl.BlockSpec((_D, _D), lambda i: (0, 0)),
        ],
        out_specs=pl.BlockSpec((_ROWS_PER_BLOCK, _D), lambda i: (i, 0)),
        out_shape=jax.ShapeDtypeStruct((_VOCAB, _D), jnp.float32),
    )(table, W)


_NC = 2
_NS = 16
_NW = _NC * _NS
_PER_W = _N // _NW
_C = 200
_ITERS = _PER_W // _C
_NBUF = 4
_LEAD = 2

_mesh = plsc.VectorSubcoreMesh(
    core_axis_name="c", subcore_axis_name="s",
    num_cores=_NC, num_subcores=_NS,
)


@functools.partial(
    pl.kernel,
    out_type=jax.ShapeDtypeStruct((_N, _D), jnp.float32),
    mesh=_mesh,
    scratch_types=[
        pltpu.VMEM((_PER_W,), jnp.int32),
        [pltpu.VMEM((_C, _D), jnp.float32) for _ in range(_NBUF)],
        [pltpu.SemaphoreType.DMA for _ in range(_NBUF)],
        [pltpu.SemaphoreType.DMA for _ in range(_NBUF)],
    ],
)
def _gather(tbl_hbm, idx_hbm, out_hbm, idx_all, rows, gsem, wsem):
    wid = lax.axis_index("s") * _NC + lax.axis_index("c")
    base = wid * _PER_W

    pltpu.sync_copy(idx_hbm.at[pl.ds(base, _PER_W)], idx_all)

    def g_copy(s, b):
        return pltpu.make_async_copy(
            tbl_hbm.at[idx_all.at[pl.ds(s * _C, _C)]], rows[b], gsem[b])

    def w_copy(s, b):
        off = pl.multiple_of(base + s * _C, _C)
        return pltpu.make_async_copy(rows[b], out_hbm.at[pl.ds(off, _C)],
                                     wsem[b])

    for b in range(_NBUF):
        g_copy(b, b).start()
        if b >= _LEAD:
            g_copy(b - _LEAD, b - _LEAD).wait()
            w_copy(b - _LEAD, b - _LEAD).start()

    def round_body(j, carry):
        for b in range(_NBUF):
            s = j * _NBUF + b
            w_copy(s - _NBUF, b).wait()
            g_copy(s, b).start()
            b2 = (b + _NBUF - _LEAD) % _NBUF
            g_copy(s - _LEAD, b2).wait()
            w_copy(s - _LEAD, b2).start()
        return carry

    lax.fori_loop(1, _ITERS // _NBUF, round_body, 0)

    for s in range(_ITERS - _LEAD, _ITERS):
        b = s % _NBUF
        g_copy(s, b).wait()
        w_copy(s, b).start()
    for s in range(_ITERS - _NBUF, _ITERS):
        w_copy(s, s % _NBUF).wait()


def kernel(x, table, W):
    P = _project(table, W)
    idx = x.reshape(_N).astype(jnp.int32)
    out = _gather(P, idx)
    return out.reshape(_B, _L, _D)


# speedup vs baseline: 1.0055x; 1.0033x over previous
import functools

import jax
import jax.numpy as jnp
from jax import lax
from jax.experimental import pallas as pl
from jax.experimental.pallas import tpu as pltpu
from jax.experimental.pallas import tpu_sc as plsc

_VOCAB = 100000
_D = 128
_B = 4096
_L = 200
_N = _B * _L


_ROWS_PER_BLOCK = 10000
_N_BLOCKS = _VOCAB // _ROWS_PER_BLOCK


def _proj_body(t_ref, w_ref, o_ref):
    o_ref[...] = lax.dot_general(
        t_ref[...], w_ref[...],
        (((1,), (1,)), ((), ())),
        preferred_element_type=jnp.float32,
    )


def _project(table, W):
    return pl.pallas_call(
        _proj_body,
        grid=(_N_BLOCKS,),
        in_specs=[
            pl.BlockSpec((_ROWS_PER_BLOCK, _D), lambda i: (i, 0)),
            pl.BlockSpec((_D, _D), lambda i: (0, 0)),
        ],
        out_specs=pl.BlockSpec((_ROWS_PER_BLOCK, _D), lambda i: (i, 0)),
        out_shape=jax.ShapeDtypeStruct((_VOCAB, _D), jnp.float32),
    )(table, W)


_NC = 2
_NS = 16
_NW = _NC * _NS
_PER_W = _N // _NW
_C = 200
_ITERS = _PER_W // _C
_NBUF = 4
_LEAD = 2

_mesh = plsc.VectorSubcoreMesh(
    core_axis_name="c", subcore_axis_name="s",
    num_cores=_NC, num_subcores=_NS,
)


@functools.partial(
    pl.kernel,
    out_type=jax.ShapeDtypeStruct((_N, _D), jnp.float32),
    mesh=_mesh,
    scratch_types=[
        pltpu.VMEM((_PER_W,), jnp.int32),
        [pltpu.VMEM((_C, _D), jnp.float32) for _ in range(_NBUF)],
        [pltpu.SemaphoreType.DMA for _ in range(_NBUF)],
        [pltpu.SemaphoreType.DMA for _ in range(_NBUF)],
    ],
)
def _gather(tbl_hbm, idx_hbm, out_hbm, idx_all, rows, gsem, wsem):
    wid = lax.axis_index("s") * _NC + lax.axis_index("c")
    base = wid * _PER_W

    pltpu.sync_copy(idx_hbm.at[pl.ds(base, _PER_W)], idx_all)

    def g_copy(s, b):
        return pltpu.make_async_copy(
            tbl_hbm.at[idx_all.at[pl.ds(s * _C, _C)]], rows[b], gsem[b])

    def w_copy(s, b):
        off = pl.multiple_of(base + s * _C, _C)
        return pltpu.make_async_copy(rows[b], out_hbm.at[pl.ds(off, _C)],
                                     wsem[b])

    for b in range(_NBUF):
        g_copy(b, b).start()
        if b >= _LEAD:
            g_copy(b - _LEAD, b - _LEAD).wait()
            w_copy(b - _LEAD, b - _LEAD).start()

    def round_body(j, carry):
        for b in range(_NBUF):
            s = j * _NBUF + b
            w_copy(s - _NBUF, b).wait()
            g_copy(s, b).start()
            b2 = (b + _NBUF - _LEAD) % _NBUF
            g_copy(s - _LEAD, b2).wait()
            w_copy(s - _LEAD, b2).start()
        return carry

    lax.fori_loop(1, _ITERS // _NBUF, round_body, 0)

    for s in range(_ITERS - _LEAD, _ITERS):
        b = s % _NBUF
        g_copy(s, b).wait()
        w_copy(s, b).start()
    for s in range(_ITERS - _NBUF, _ITERS):
        w_copy(s, s % _NBUF).wait()


def kernel(x, table, W):
    P = _project(table, W)
    idx = x.reshape(_N).astype(jnp.int32)
    out = _gather(P, idx)
    return out.reshape(_B, _L, _D)
